# Initial kernel scaffold; baseline (speedup 1.0000x reference)
import jax
import jax.numpy as jnp
from jax.experimental import pallas as pl


def kernel(user_id, movie_title, user_table, movie_table, W1, b1, W2, b2, W3, b3):
    raise NotImplementedError("write your pallas kernel here")


# speedup vs baseline: 1.2948x; 1.2948x over previous
import functools

import jax
import jax.numpy as jnp
from jax import lax
from jax.experimental import pallas as pl
from jax.experimental.pallas import tpu as pltpu
from jax.experimental.pallas import tpu_sc as plsc

_B = 16384
_D = 32
_NC = 2
_NS = 16
_NW = _NC * _NS
_BPW = _B // _NW
_CHUNK = 128
_NCHUNK = _BPW // _CHUNK

_BB = 2048


def _gather_body(uid_ref, mid_ref, utab_ref, mtab_ref, uout_ref, mout_ref,
                 uidx_v, midx_v, urows_v, mrows_v, sem):
    wid = lax.axis_index("s") * _NC + lax.axis_index("c")
    base = wid * _BPW
    row0 = wid * _NCHUNK
    pltpu.sync_copy(uid_ref.at[pl.ds(row0, _NCHUNK)], uidx_v)
    pltpu.sync_copy(mid_ref.at[pl.ds(row0, _NCHUNK)], midx_v)
    copies = []
    for j in range(_NCHUNK):
        dst = pl.ds(j * _CHUNK, _CHUNK)
        copies.append(pltpu.async_copy(utab_ref.at[uidx_v.at[j]],
                                       urows_v.at[dst], sem))
        copies.append(pltpu.async_copy(mtab_ref.at[midx_v.at[j]],
                                       mrows_v.at[dst], sem))
    for c in copies:
        c.wait()
    pltpu.sync_copy(urows_v, uout_ref.at[pl.ds(base, _BPW)])
    pltpu.sync_copy(mrows_v, mout_ref.at[pl.ds(base, _BPW)])


_gather = pl.kernel(
    _gather_body,
    out_type=(jax.ShapeDtypeStruct((_B, _D), jnp.float32),
              jax.ShapeDtypeStruct((_B, _D), jnp.float32)),
    mesh=plsc.VectorSubcoreMesh(core_axis_name="c", subcore_axis_name="s"),
    scratch_types=[
        pltpu.VMEM((_NCHUNK, _CHUNK), jnp.int32),
        pltpu.VMEM((_NCHUNK, _CHUNK), jnp.int32),
        pltpu.VMEM((_BPW, _D), jnp.float32),
        pltpu.VMEM((_BPW, _D), jnp.float32),
        pltpu.SemaphoreType.DMA,
    ],
    compiler_params=pltpu.CompilerParams(use_tc_tiling_on_sc=False),
)


def _mlp_body(u_ref, m_ref, w1u_ref, w1m_ref, b1_ref, w2_ref, b2_ref,
              w3_ref, b3_ref, out_ref):
    x = (jnp.dot(u_ref[...], w1u_ref[...], preferred_element_type=jnp.float32)
         + jnp.dot(m_ref[...], w1m_ref[...], preferred_element_type=jnp.float32)
         + b1_ref[...])
    h1 = jnp.maximum(x, 0.0)
    h2 = jnp.maximum(
        jnp.dot(h1, w2_ref[...], preferred_element_type=jnp.float32)
        + b2_ref[...], 0.0)
    out_ref[...] = (jnp.dot(h2, w3_ref[...], preferred_element_type=jnp.float32)
                    + b3_ref[...])


_mlp = pl.pallas_call(
    _mlp_body,
    grid=(_B // _BB,),
    in_specs=[
        pl.BlockSpec((_BB, _D), lambda i: (i, 0)),
        pl.BlockSpec((_BB, _D), lambda i: (i, 0)),
        pl.BlockSpec((_D, 256), lambda i: (0, 0)),
        pl.BlockSpec((_D, 256), lambda i: (0, 0)),
        pl.BlockSpec((1, 256), lambda i: (0, 0)),
        pl.BlockSpec((256, 64), lambda i: (0, 0)),
        pl.BlockSpec((1, 64), lambda i: (0, 0)),
        pl.BlockSpec((64, 1), lambda i: (0, 0)),
        pl.BlockSpec((1, 1), lambda i: (0, 0)),
    ],
    out_specs=pl.BlockSpec((_BB, 1), lambda i: (i, 0)),
    out_shape=jax.ShapeDtypeStruct((_B, 1), jnp.float32),
)


@jax.jit
def kernel(user_id, movie_title, user_table, movie_table,
           W1, b1, W2, b2, W3, b3):
    uid2d = user_id.astype(jnp.int32).reshape(_NW * _NCHUNK, _CHUNK)
    mid2d = movie_title.astype(jnp.int32).reshape(_NW * _NCHUNK, _CHUNK)
    u_emb, m_emb = _gather(uid2d, mid2d, user_table, movie_table)
    return _mlp(u_emb, m_emb,
                W1[:_D], W1[_D:],
                b1.reshape(1, 256),
                W2, b2.reshape(1, 64),
                W3, b3.reshape(1, 1))


# speedup vs baseline: 1.4164x; 1.0939x over previous
import jax
import jax.numpy as jnp
from jax import lax
from jax.experimental import pallas as pl
from jax.experimental.pallas import tpu as pltpu
from jax.experimental.pallas import tpu_sc as plsc

_B = 16384
_D = 32
_NC = 2
_NS = 16
_NW = _NC * _NS
_BPW = _B // _NW
_CHUNK = 128
_NCHUNK = _BPW // _CHUNK

_BB = 2048


def _gather_body(uid_ref, mid_ref, utab_ref, mtab_ref, uout_ref, mout_ref,
                 uidx_v, midx_v, urows_v, mrows_v, sem):
    wid = lax.axis_index("s") * _NC + lax.axis_index("c")
    base = wid * _BPW
    pltpu.sync_copy(uid_ref.at[pl.ds(base, _BPW)], uidx_v)
    pltpu.sync_copy(mid_ref.at[pl.ds(base, _BPW)], midx_v)
    copies = []
    for j in range(_NCHUNK):
        src = pl.ds(j * _CHUNK, _CHUNK)
        copies.append(pltpu.async_copy(utab_ref.at[uidx_v.at[src]],
                                       urows_v.at[src], sem))
        copies.append(pltpu.async_copy(mtab_ref.at[midx_v.at[src]],
                                       mrows_v.at[src], sem))
    for c in copies:
        c.wait()
    dst = (pl.ds(base, _BPW), pl.ds(0, _D))
    pltpu.sync_copy(urows_v, uout_ref.at[dst])
    pltpu.sync_copy(mrows_v, mout_ref.at[dst])


_gather = pl.kernel(
    _gather_body,
    out_type=(jax.ShapeDtypeStruct((_B, 128), jnp.float32),
              jax.ShapeDtypeStruct((_B, 128), jnp.float32)),
    mesh=plsc.VectorSubcoreMesh(core_axis_name="c", subcore_axis_name="s"),
    scratch_types=[
        pltpu.VMEM((_BPW,), jnp.int32),
        pltpu.VMEM((_BPW,), jnp.int32),
        pltpu.VMEM((_BPW, _D), jnp.float32),
        pltpu.VMEM((_BPW, _D), jnp.float32),
        pltpu.SemaphoreType.DMA,
    ],
    compiler_params=pltpu.CompilerParams(use_tc_tiling_on_sc=False),
)


def _mlp_body(u_ref, m_ref, w1_ref, b1_ref, w2_ref, b2_ref,
              w3_ref, b3_ref, out_ref):
    x = (jnp.dot(u_ref[:, :_D], w1_ref[:_D], preferred_element_type=jnp.float32)
         + jnp.dot(m_ref[:, :_D], w1_ref[_D:], preferred_element_type=jnp.float32)
         + b1_ref[...][None, :])
    h1 = jnp.maximum(x, 0.0)
    h2 = jnp.maximum(
        jnp.dot(h1, w2_ref[...], preferred_element_type=jnp.float32)
        + b2_ref[...][None, :], 0.0)
    out_ref[...] = (jnp.dot(h2, w3_ref[...], preferred_element_type=jnp.float32)
                    + b3_ref[...][None, :])


_mlp = pl.pallas_call(
    _mlp_body,
    grid=(_B // _BB,),
    in_specs=[
        pl.BlockSpec((_BB, 128), lambda i: (i, 0)),
        pl.BlockSpec((_BB, 128), lambda i: (i, 0)),
        pl.BlockSpec((2 * _D, 256), lambda i: (0, 0)),
        pl.BlockSpec((256,), lambda i: (0,)),
        pl.BlockSpec((256, 64), lambda i: (0, 0)),
        pl.BlockSpec((64,), lambda i: (0,)),
        pl.BlockSpec((64, 1), lambda i: (0, 0)),
        pl.BlockSpec((1,), lambda i: (0,)),
    ],
    out_specs=pl.BlockSpec((_BB, 1), lambda i: (i, 0)),
    out_shape=jax.ShapeDtypeStruct((_B, 1), jnp.float32),
)


@jax.jit
def kernel(user_id, movie_title, user_table, movie_table,
           W1, b1, W2, b2, W3, b3):
    u_emb, m_emb = _gather(user_id.astype(jnp.int32),
                           movie_title.astype(jnp.int32),
                           user_table, movie_table)
    return _mlp(u_emb, m_emb, W1, b1, W2, b2, W3, b3)


# speedup vs baseline: 1.4850x; 1.0485x over previous
import jax
import jax.numpy as jnp
from jax import lax
from jax.experimental import pallas as pl
from jax.experimental.pallas import tpu as pltpu
from jax.experimental.pallas import tpu_sc as plsc

_B = 16384
_D = 32
_NC = 2
_NS = 16
_NW = _NC * _NS
_BPW = _B // _NW
_CHUNK = 128
_NCHUNK = _BPW // _CHUNK

_BB = 2048


def _gather_body(uid_ref, mid_ref, utab_ref, mtab_ref, xout_ref,
                 uidx_v, midx_v, urows_v, mrows_v, sem):
    wid = lax.axis_index("s") * _NC + lax.axis_index("c")
    base = wid * _BPW
    pltpu.sync_copy(uid_ref.at[pl.ds(base, _BPW)], uidx_v)
    pltpu.sync_copy(mid_ref.at[pl.ds(base, _BPW)], midx_v)
    copies = []
    for j in range(_NCHUNK):
        sl = pl.ds(j * _CHUNK, _CHUNK)
        for d in range(_D):
            copies.append(pltpu.async_copy(
                utab_ref.at[d].at[uidx_v.at[sl]], urows_v.at[d, sl], sem))
            copies.append(pltpu.async_copy(
                mtab_ref.at[d].at[midx_v.at[sl]], mrows_v.at[d, sl], sem))
    for c in copies:
        c.wait()
    pltpu.sync_copy(urows_v, xout_ref.at[pl.ds(0, _D), pl.ds(base, _BPW)])
    pltpu.sync_copy(mrows_v, xout_ref.at[pl.ds(_D, _D), pl.ds(base, _BPW)])


_gather = pl.kernel(
    _gather_body,
    out_type=jax.ShapeDtypeStruct((2 * _D, _B), jnp.float32),
    mesh=plsc.VectorSubcoreMesh(core_axis_name="c", subcore_axis_name="s"),
    scratch_types=[
        pltpu.VMEM((_BPW,), jnp.int32),
        pltpu.VMEM((_BPW,), jnp.int32),
        pltpu.VMEM((_D, _BPW), jnp.float32),
        pltpu.VMEM((_D, _BPW), jnp.float32),
        pltpu.SemaphoreType.DMA,
    ],
    compiler_params=pltpu.CompilerParams(use_tc_tiling_on_sc=False),
)


def _mlp_body(x_ref, w1_ref, b1_ref, w2_ref, b2_ref, w3_ref, b3_ref, out_ref):
    cdims = (((0,), (0,)), ((), ()))
    h1 = jnp.maximum(
        lax.dot_general(w1_ref[...], x_ref[...], cdims,
                        preferred_element_type=jnp.float32)
        + b1_ref[...][:, None], 0.0)
    h2 = jnp.maximum(
        lax.dot_general(w2_ref[...], h1, cdims,
                        preferred_element_type=jnp.float32)
        + b2_ref[...][:, None], 0.0)
    out_ref[...] = (
        lax.dot_general(w3_ref[...], h2, cdims,
                        preferred_element_type=jnp.float32)
        + b3_ref[...][:, None])


_mlp = pl.pallas_call(
    _mlp_body,
    grid=(_B // _BB,),
    in_specs=[
        pl.BlockSpec((2 * _D, _BB), lambda i: (0, i)),
        pl.BlockSpec((2 * _D, 256), lambda i: (0, 0)),
        pl.BlockSpec((256,), lambda i: (0,)),
        pl.BlockSpec((256, 64), lambda i: (0, 0)),
        pl.BlockSpec((64,), lambda i: (0,)),
        pl.BlockSpec((64, 1), lambda i: (0, 0)),
        pl.BlockSpec((1,), lambda i: (0,)),
    ],
    out_specs=pl.BlockSpec((1, _BB), lambda i: (0, i)),
    out_shape=jax.ShapeDtypeStruct((1, _B), jnp.float32),
)


@jax.jit
def kernel(user_id, movie_title, user_table, movie_table,
           W1, b1, W2, b2, W3, b3):
    x_t = _gather(user_id.astype(jnp.int32), movie_title.astype(jnp.int32),
                  user_table.T, movie_table.T)
    out_t = _mlp(x_t, W1, b1, W2, b2, W3, b3)
    return out_t.T


# speedup vs baseline: 1.6360x; 1.1016x over previous
import jax
import jax.numpy as jnp
from jax import lax
from jax.experimental import pallas as pl
from jax.experimental.pallas import tpu as pltpu
from jax.experimental.pallas import tpu_sc as plsc

_B = 16384
_D = 32
_NC = 2
_NS = 16
_NW = _NC * _NS
_BPW = _B // _NW
_CHUNK = 128
_NCHUNK = _BPW // _CHUNK


def _gather_body(idx_ref, tab_ref, xout_ref, idx_v, rows_v, sem):
    wid = lax.axis_index("s") * _NC + lax.axis_index("c")
    base = wid * _BPW
    pltpu.sync_copy(idx_ref.at[pl.ds(base, _BPW)], idx_v)
    copies = []
    for j in range(_NCHUNK):
        sl = pl.ds(j * _CHUNK, _CHUNK)
        for d in range(_D):
            copies.append(pltpu.async_copy(
                tab_ref.at[d].at[idx_v.at[sl]], rows_v.at[d, sl], sem))
    for c in copies:
        c.wait()
    pltpu.sync_copy(rows_v, xout_ref.at[:, pl.ds(base, _BPW)])


def _make_gather():
    return pl.kernel(
        _gather_body,
        out_type=jax.ShapeDtypeStruct((_D, _B), jnp.float32),
        mesh=plsc.VectorSubcoreMesh(core_axis_name="c", subcore_axis_name="s"),
        scratch_types=[
            pltpu.VMEM((_BPW,), jnp.int32),
            pltpu.VMEM((_D, _BPW), jnp.float32),
            pltpu.SemaphoreType.DMA,
        ],
        compiler_params=pltpu.CompilerParams(use_tc_tiling_on_sc=False),
    )


_gather_u = _make_gather()
_gather_m = _make_gather()


def _mlp_body(xu_ref, xm_ref, w1_ref, b1_ref, w2_ref, b2_ref,
              w3_ref, b3_ref, out_ref):
    cdims = (((0,), (0,)), ((), ()))
    h1 = jnp.maximum(
        lax.dot_general(w1_ref[:_D], xu_ref[...], cdims,
                        preferred_element_type=jnp.float32)
        + lax.dot_general(w1_ref[_D:], xm_ref[...], cdims,
                          preferred_element_type=jnp.float32)
        + b1_ref[...][:, None], 0.0)
    h2 = jnp.maximum(
        lax.dot_general(w2_ref[...], h1, cdims,
                        preferred_element_type=jnp.float32)
        + b2_ref[...][:, None], 0.0)
    out_ref[...] = (
        lax.dot_general(w3_ref[...], h2, cdims,
                        preferred_element_type=jnp.float32)
        + b3_ref[...][:, None])


_BB = 2048

_mlp = pl.pallas_call(
    _mlp_body,
    grid=(_B // _BB,),
    in_specs=[
        pl.BlockSpec((_D, _BB), lambda i: (0, i)),
        pl.BlockSpec((_D, _BB), lambda i: (0, i)),
        pl.BlockSpec((2 * _D, 256), lambda i: (0, 0)),
        pl.BlockSpec((256,), lambda i: (0,)),
        pl.BlockSpec((256, 64), lambda i: (0, 0)),
        pl.BlockSpec((64,), lambda i: (0,)),
        pl.BlockSpec((64, 1), lambda i: (0, 0)),
        pl.BlockSpec((1,), lambda i: (0,)),
    ],
    out_specs=pl.BlockSpec((1, _BB), lambda i: (0, i)),
    out_shape=jax.ShapeDtypeStruct((1, _B), jnp.float32),
)


@jax.jit
def kernel(user_id, movie_title, user_table, movie_table,
           W1, b1, W2, b2, W3, b3):
    xu = _gather_u(user_id.astype(jnp.int32), user_table.T)
    xm = _gather_m(movie_title.astype(jnp.int32), movie_table.T)
    out_t = _mlp(xu, xm, W1, b1, W2, b2, W3, b3)
    return out_t.T


# speedup vs baseline: 1.7003x; 1.0393x over previous
import jax
import jax.numpy as jnp
from jax import lax
from jax.experimental import pallas as pl
from jax.experimental.pallas import tpu as pltpu
from jax.experimental.pallas import tpu_sc as plsc

_B = 16384
_D = 32
_NC = 2
_NS = 16
_NW = _NC * _NS
_BPW = _B // _NW
_CHUNK = 128
_NCHUNK = _BPW // _CHUNK


def _gather_body(idx_ref, tab_ref, xout_ref, idx_v, rows_v, sem):
    wid = lax.axis_index("s") * _NC + lax.axis_index("c")
    base = wid * _BPW
    pltpu.sync_copy(idx_ref.at[pl.ds(base, _BPW)], idx_v)
    copies = []
    for d in range(_D):
        copies.append(pltpu.async_copy(
            tab_ref.at[d].at[idx_v], rows_v.at[d], sem))
    for c in copies:
        c.wait()
    pltpu.sync_copy(rows_v, xout_ref.at[:, pl.ds(base, _BPW)])


def _make_gather():
    return pl.kernel(
        _gather_body,
        out_type=jax.ShapeDtypeStruct((_D, _B), jnp.float32),
        mesh=plsc.VectorSubcoreMesh(core_axis_name="c", subcore_axis_name="s"),
        scratch_types=[
            pltpu.VMEM((_BPW,), jnp.int32),
            pltpu.VMEM((_D, _BPW), jnp.float32),
            pltpu.SemaphoreType.DMA,
        ],
        compiler_params=pltpu.CompilerParams(use_tc_tiling_on_sc=False),
    )


_gather_u = _make_gather()
_gather_m = _make_gather()


def _mlp_body(xu_ref, xm_ref, w1_ref, b1_ref, w2_ref, b2_ref,
              w3_ref, b3_ref, out_ref):
    cdims = (((0,), (0,)), ((), ()))
    h1 = jnp.maximum(
        lax.dot_general(w1_ref[:_D], xu_ref[...], cdims,
                        preferred_element_type=jnp.float32)
        + lax.dot_general(w1_ref[_D:], xm_ref[...], cdims,
                          preferred_element_type=jnp.float32)
        + b1_ref[...][:, None], 0.0)
    h2 = jnp.maximum(
        lax.dot_general(w2_ref[...], h1, cdims,
                        preferred_element_type=jnp.float32)
        + b2_ref[...][:, None], 0.0)
    out_ref[...] = (
        lax.dot_general(w3_ref[...], h2, cdims,
                        preferred_element_type=jnp.float32)
        + b3_ref[...][:, None])


_BB = 2048

_mlp = pl.pallas_call(
    _mlp_body,
    grid=(_B // _BB,),
    in_specs=[
        pl.BlockSpec((_D, _BB), lambda i: (0, i)),
        pl.BlockSpec((_D, _BB), lambda i: (0, i)),
        pl.BlockSpec((2 * _D, 256), lambda i: (0, 0)),
        pl.BlockSpec((256,), lambda i: (0,)),
        pl.BlockSpec((256, 64), lambda i: (0, 0)),
        pl.BlockSpec((64,), lambda i: (0,)),
        pl.BlockSpec((64, 1), lambda i: (0, 0)),
        pl.BlockSpec((1,), lambda i: (0,)),
    ],
    out_specs=pl.BlockSpec((1, _BB), lambda i: (0, i)),
    out_shape=jax.ShapeDtypeStruct((1, _B), jnp.float32),
)


@jax.jit
def kernel(user_id, movie_title, user_table, movie_table,
           W1, b1, W2, b2, W3, b3):
    xu = _gather_u(user_id.astype(jnp.int32), user_table.T)
    xm = _gather_m(movie_title.astype(jnp.int32), movie_table.T)
    out_t = _mlp(xu, xm, W1, b1, W2, b2, W3, b3)
    return out_t.T


# speedup vs baseline: 1.9006x; 1.1178x over previous
import jax
import jax.numpy as jnp
from jax import lax
from jax.experimental import pallas as pl
from jax.experimental.pallas import tpu as pltpu
from jax.experimental.pallas import tpu_sc as plsc

_B = 16384
_D = 32
_V = 100001
_NC = 2
_NS = 16
_NW = _NC * _NS
_BPW = _B // _NW
_CHUNK = 128
_NCHUNK = _BPW // _CHUNK


def _gather_body(idx_ref, tab_ref, xout_ref, idx_v, fidx_v, rows_v, sem):
    wid = lax.axis_index("s") * _NC + lax.axis_index("c")
    base = wid * _BPW
    pltpu.sync_copy(idx_ref.at[pl.ds(base, _BPW)], idx_v)
    for k in range(_BPW // 16):
        sl = pl.ds(16 * k, 16)
        v = idx_v[sl]
        for d in range(_D):
            fidx_v[d, sl] = v + jnp.int32(d * _V)
    copies = []
    for d in range(_D):
        copies.append(pltpu.async_copy(
            tab_ref.at[fidx_v.at[d]], rows_v.at[d], sem))
    for c in copies:
        c.wait()
    pltpu.sync_copy(rows_v, xout_ref.at[:, pl.ds(base, _BPW)])


def _make_gather():
    return pl.kernel(
        _gather_body,
        out_type=jax.ShapeDtypeStruct((_D, _B), jnp.float32),
        mesh=plsc.VectorSubcoreMesh(core_axis_name="c", subcore_axis_name="s"),
        scratch_types=[
            pltpu.VMEM((_BPW,), jnp.int32),
            pltpu.VMEM((_D, _BPW), jnp.int32),
            pltpu.VMEM((_D, _BPW), jnp.float32),
            pltpu.SemaphoreType.DMA,
        ],
        compiler_params=pltpu.CompilerParams(use_tc_tiling_on_sc=False),
    )


_gather_u = _make_gather()
_gather_m = _make_gather()


def _mlp_body(xu_ref, xm_ref, w1_ref, b1_ref, w2_ref, b2_ref,
              w3_ref, b3_ref, out_ref):
    cdims = (((0,), (0,)), ((), ()))
    h1 = jnp.maximum(
        lax.dot_general(w1_ref[:_D], xu_ref[...], cdims,
                        preferred_element_type=jnp.float32)
        + lax.dot_general(w1_ref[_D:], xm_ref[...], cdims,
                          preferred_element_type=jnp.float32)
        + b1_ref[...][:, None], 0.0)
    h2 = jnp.maximum(
        lax.dot_general(w2_ref[...], h1, cdims,
                        preferred_element_type=jnp.float32)
        + b2_ref[...][:, None], 0.0)
    out_ref[...] = (
        lax.dot_general(w3_ref[...], h2, cdims,
                        preferred_element_type=jnp.float32)
        + b3_ref[...][:, None])


_BB = 2048

_mlp = pl.pallas_call(
    _mlp_body,
    grid=(_B // _BB,),
    in_specs=[
        pl.BlockSpec((_D, _BB), lambda i: (0, i)),
        pl.BlockSpec((_D, _BB), lambda i: (0, i)),
        pl.BlockSpec((2 * _D, 256), lambda i: (0, 0)),
        pl.BlockSpec((256,), lambda i: (0,)),
        pl.BlockSpec((256, 64), lambda i: (0, 0)),
        pl.BlockSpec((64,), lambda i: (0,)),
        pl.BlockSpec((64, 1), lambda i: (0, 0)),
        pl.BlockSpec((1,), lambda i: (0,)),
    ],
    out_specs=pl.BlockSpec((1, _BB), lambda i: (0, i)),
    out_shape=jax.ShapeDtypeStruct((1, _B), jnp.float32),
)


@jax.jit
def kernel(user_id, movie_title, user_table, movie_table,
           W1, b1, W2, b2, W3, b3):
    xu = _gather_u(user_id.astype(jnp.int32), user_table.T.reshape(-1))
    xm = _gather_m(movie_title.astype(jnp.int32), movie_table.T.reshape(-1))
    out_t = _mlp(xu, xm, W1, b1, W2, b2, W3, b3)
    return out_t.T
